# baseline (speedup 1.0000x reference)
import jax
import jax.numpy as jnp
from jax.experimental import pallas as pl

_COMMITMENT_COST = 0.25
_NT = 256


def _vq_body(z_ref, cb_ref, zq_ref, idx_ref, ls_ref):
    k = cb_ref.shape[0]
    nt = z_ref.shape[2]
    cb = cb_ref[...]
    zt = z_ref[0]
    s = jax.lax.dot_general(
        cb.astype(jnp.bfloat16), zt.astype(jnp.bfloat16),
        (((1,), (0,)), ((), ())),
        preferred_element_type=jnp.float32)
    z2 = jnp.sum(zt * zt, axis=0, keepdims=True)
    c2 = jnp.sum(cb * cb, axis=1, keepdims=True)
    dist = z2 - 2.0 * s + c2
    mins = jnp.min(dist, axis=0, keepdims=True)
    kiota = jax.lax.broadcasted_iota(jnp.int32, (k, nt), 0)
    idx = jnp.min(jnp.where(dist == mins, kiota, k),
                  axis=0, keepdims=True)
    idx_ref[0] = idx
    onehot = (kiota == idx).astype(jnp.float32)
    zq = jax.lax.dot_general(
        cb, onehot, (((0,), (0,)), ((), ())),
        preferred_element_type=jnp.float32,
        precision=jax.lax.Precision.HIGHEST)
    zq_ref[0] = zq

    @pl.when(pl.program_id(0) == 0)
    def _init():
        ls_ref[...] = jnp.zeros_like(ls_ref)

    diff = zq - zt
    ls_ref[...] += jnp.sum(diff * diff, axis=0, keepdims=True)


def kernel(z, codebook):
    b, d, h, w = z.shape
    kk = codebook.shape[0]
    hw = h * w
    nt = min(_NT, hw)
    nblk = hw // nt
    grid = b * nblk
    z3 = z.reshape(b, d, hw)
    zq3, idx, ls = pl.pallas_call(
        _vq_body,
        grid=(grid,),
        in_specs=[
            pl.BlockSpec((1, d, nt), lambda i: (i // nblk, 0, i % nblk)),
            pl.BlockSpec((kk, d), lambda i: (0, 0)),
        ],
        out_specs=[
            pl.BlockSpec((1, d, nt), lambda i: (i // nblk, 0, i % nblk)),
            pl.BlockSpec((1, 1, nt), lambda i: (i, 0, 0)),
            pl.BlockSpec((1, nt), lambda i: (0, 0)),
        ],
        out_shape=[
            jax.ShapeDtypeStruct((b, d, hw), jnp.float32),
            jax.ShapeDtypeStruct((grid, 1, nt), jnp.int32),
            jax.ShapeDtypeStruct((1, nt), jnp.float32),
        ],
    )(z3, codebook)
    zq_out = zq3.reshape(b, d, h, w)
    idx_out = idx.reshape(b, h, w)
    mse = jnp.sum(ls) / (b * d * hw)
    vq_loss = mse + _COMMITMENT_COST * mse
    return zq_out, idx_out, vq_loss


# speedup vs baseline: 1.4813x; 1.4813x over previous
import jax
import jax.numpy as jnp
from jax.experimental import pallas as pl

_COMMITMENT_COST = 0.25
_NT = 256


def _vq_body(z_ref, cb_ref, zq_ref, idx_ref, ls_ref):
    k = cb_ref.shape[0]
    nt = z_ref.shape[2]
    cb = cb_ref[...]
    zt = z_ref[0]
    cbm2 = (-2.0 * cb).astype(jnp.bfloat16)
    s_neg = jax.lax.dot_general(
        cbm2, zt.astype(jnp.bfloat16),
        (((1,), (0,)), ((), ())),
        preferred_element_type=jnp.float32)
    z2 = jnp.sum(zt * zt, axis=0, keepdims=True)
    c2 = jnp.sum(cb * cb, axis=1, keepdims=True)
    dist = (z2 + s_neg) + c2
    mins = jnp.min(dist, axis=0, keepdims=True)
    kiota = jax.lax.broadcasted_iota(jnp.int32, (k, nt), 0)
    idx = jnp.min(jnp.where(dist == mins, kiota, k),
                  axis=0, keepdims=True)
    idx_ref[0] = idx
    kiota16 = jax.lax.broadcasted_iota(jnp.int16, (k, nt), 0)
    onehot = jnp.where(kiota16 == idx.astype(jnp.int16),
                       jnp.bfloat16(1), jnp.bfloat16(0))
    cb_hi = cb.astype(jnp.bfloat16)
    cb_lo = (cb - cb_hi.astype(jnp.float32)).astype(jnp.bfloat16)
    dims = (((0,), (0,)), ((), ()))
    zq = (jax.lax.dot_general(cb_hi, onehot, dims,
                              preferred_element_type=jnp.float32)
          + jax.lax.dot_general(cb_lo, onehot, dims,
                                preferred_element_type=jnp.float32))
    zq_ref[0] = zq

    @pl.when(pl.program_id(0) == 0)
    def _init():
        ls_ref[...] = jnp.zeros_like(ls_ref)

    diff = zq - zt
    ls_ref[...] += jnp.sum(diff * diff, axis=0, keepdims=True)


def kernel(z, codebook):
    b, d, h, w = z.shape
    kk = codebook.shape[0]
    hw = h * w
    nt = min(_NT, hw)
    nblk = hw // nt
    grid = b * nblk
    z3 = z.reshape(b, d, hw)
    zq3, idx, ls = pl.pallas_call(
        _vq_body,
        grid=(grid,),
        in_specs=[
            pl.BlockSpec((1, d, nt), lambda i: (i // nblk, 0, i % nblk)),
            pl.BlockSpec((kk, d), lambda i: (0, 0)),
        ],
        out_specs=[
            pl.BlockSpec((1, d, nt), lambda i: (i // nblk, 0, i % nblk)),
            pl.BlockSpec((1, 1, nt), lambda i: (i, 0, 0)),
            pl.BlockSpec((1, nt), lambda i: (0, 0)),
        ],
        out_shape=[
            jax.ShapeDtypeStruct((b, d, hw), jnp.float32),
            jax.ShapeDtypeStruct((grid, 1, nt), jnp.int32),
            jax.ShapeDtypeStruct((1, nt), jnp.float32),
        ],
    )(z3, codebook)
    zq_out = zq3.reshape(b, d, h, w)
    idx_out = idx.reshape(b, h, w)
    mse = jnp.sum(ls) / (b * d * hw)
    vq_loss = mse + _COMMITMENT_COST * mse
    return zq_out, idx_out, vq_loss


# speedup vs baseline: 1.5087x; 1.0185x over previous
import jax
import jax.numpy as jnp
from jax.experimental import pallas as pl

_COMMITMENT_COST = 0.25
_NT = 256


def _prep_body(cb_ref, cbm2_ref, cbhi_ref, cblo_ref, c2_ref):
    cb = cb_ref[...]
    cbm2_ref[...] = (-2.0 * cb).astype(jnp.bfloat16)
    hi = cb.astype(jnp.bfloat16)
    cbhi_ref[...] = hi
    cblo_ref[...] = (cb - hi.astype(jnp.float32)).astype(jnp.bfloat16)
    c2_ref[...] = jnp.sum(cb * cb, axis=1, keepdims=True)


def _vq_body(z_ref, cbm2_ref, cbhi_ref, cblo_ref, c2_ref,
             zq_ref, idx_ref, ls_ref):
    k = cbm2_ref.shape[0]
    nt = z_ref.shape[2]
    zt = z_ref[0]
    s_neg = jax.lax.dot_general(
        cbm2_ref[...], zt.astype(jnp.bfloat16),
        (((1,), (0,)), ((), ())),
        preferred_element_type=jnp.float32)
    z2 = jnp.sum(zt * zt, axis=0, keepdims=True)
    dist = (z2 + s_neg) + c2_ref[...]
    mins = jnp.min(dist, axis=0, keepdims=True)
    kiota = jax.lax.broadcasted_iota(jnp.int32, (k, nt), 0)
    idx = jnp.min(jnp.where(dist == mins, kiota, k),
                  axis=0, keepdims=True)
    idx_ref[0] = idx
    kiota16 = jax.lax.broadcasted_iota(jnp.int16, (k, nt), 0)
    onehot = jnp.where(kiota16 == idx.astype(jnp.int16),
                       jnp.bfloat16(1), jnp.bfloat16(0))
    dims = (((0,), (0,)), ((), ()))
    zq = (jax.lax.dot_general(cbhi_ref[...], onehot, dims,
                              preferred_element_type=jnp.float32)
          + jax.lax.dot_general(cblo_ref[...], onehot, dims,
                                preferred_element_type=jnp.float32))
    zq_ref[0] = zq

    @pl.when(pl.program_id(0) == 0)
    def _init():
        ls_ref[...] = jnp.zeros_like(ls_ref)

    diff = zq - zt
    ls_ref[...] += jnp.sum(diff * diff, axis=0, keepdims=True)


def kernel(z, codebook):
    b, d, h, w = z.shape
    kk = codebook.shape[0]
    hw = h * w
    nt = min(_NT, hw)
    nblk = hw // nt
    grid = b * nblk
    z3 = z.reshape(b, d, hw)

    cbm2, cbhi, cblo, c2 = pl.pallas_call(
        _prep_body,
        out_shape=[
            jax.ShapeDtypeStruct((kk, d), jnp.bfloat16),
            jax.ShapeDtypeStruct((kk, d), jnp.bfloat16),
            jax.ShapeDtypeStruct((kk, d), jnp.bfloat16),
            jax.ShapeDtypeStruct((kk, 1), jnp.float32),
        ],
    )(codebook)

    zq3, idx, ls = pl.pallas_call(
        _vq_body,
        grid=(grid,),
        in_specs=[
            pl.BlockSpec((1, d, nt), lambda i: (i // nblk, 0, i % nblk)),
            pl.BlockSpec((kk, d), lambda i: (0, 0)),
            pl.BlockSpec((kk, d), lambda i: (0, 0)),
            pl.BlockSpec((kk, d), lambda i: (0, 0)),
            pl.BlockSpec((kk, 1), lambda i: (0, 0)),
        ],
        out_specs=[
            pl.BlockSpec((1, d, nt), lambda i: (i // nblk, 0, i % nblk)),
            pl.BlockSpec((1, 1, nt), lambda i: (i, 0, 0)),
            pl.BlockSpec((1, nt), lambda i: (0, 0)),
        ],
        out_shape=[
            jax.ShapeDtypeStruct((b, d, hw), jnp.float32),
            jax.ShapeDtypeStruct((grid, 1, nt), jnp.int32),
            jax.ShapeDtypeStruct((1, nt), jnp.float32),
        ],
    )(z3, cbm2, cbhi, cblo, c2)
    zq_out = zq3.reshape(b, d, h, w)
    idx_out = idx.reshape(b, h, w)
    mse = jnp.sum(ls) / (b * d * hw)
    vq_loss = mse + _COMMITMENT_COST * mse
    return zq_out, idx_out, vq_loss
